# baseline (speedup 1.0000x reference)
import functools

import jax
import jax.numpy as jnp
from jax import lax
from jax.experimental import pallas as pl
from jax.experimental.pallas import tpu as pltpu
from jax.experimental.pallas import tpu_sc as plsc

N = 10000
D = 128
D_HID = 256
NPAD = 10240
NC = 2
NS = 16
NW = NC * NS
CHUNK = 128
NBUF = 2
BM = 512


def _make_spmm(e_pad, with_cnt):
  per_worker = e_pad // NW
  n_chunks = per_worker // CHUNK
  assert n_chunks % NBUF == 0 and n_chunks >= 2 * NBUF
  rps = NPAD // NS

  mesh = plsc.VectorSubcoreMesh(core_axis_name="c", subcore_axis_name="s")
  out_type = [jax.ShapeDtypeStruct((NC, NPAD, D), jnp.float32)]
  scratch = (
      [pltpu.VMEM_SHARED((NPAD, D), jnp.float32)] +
      [pltpu.VMEM((CHUNK,), jnp.int32)] * NBUF +
      [pltpu.VMEM((CHUNK,), jnp.int32)] * NBUF +
      [pltpu.VMEM((CHUNK, D), jnp.float32)] * NBUF +
      [pltpu.SemaphoreType.DMA] * (3 * NBUF)
  )
  if with_cnt:
    out_type.append(jax.ShapeDtypeStruct((NC, NPAD), jnp.float32))
    scratch.append(pltpu.VMEM_SHARED((NPAD,), jnp.float32))
    scratch.append(pltpu.VMEM((CHUNK,), jnp.float32))

  def body(*refs):
    if with_cnt:
      h_hbm, row_hbm, col_hbm, zr_hbm, zc_hbm, p_hbm, cnt_hbm = refs[:7]
      rest = refs[7:]
    else:
      h_hbm, row_hbm, col_hbm, zr_hbm, p_hbm = refs[:5]
      rest = refs[5:]
    acc = rest[0]
    ridx = rest[1:1 + NBUF]
    cidx = rest[1 + NBUF:1 + 2 * NBUF]
    rows = rest[1 + 2 * NBUF:1 + 3 * NBUF]
    gsem = rest[1 + 3 * NBUF:1 + 4 * NBUF]
    ssem = rest[1 + 4 * NBUF:1 + 5 * NBUF]
    osem = rest[1 + 5 * NBUF:1 + 6 * NBUF]
    if with_cnt:
      cacc, ones = rest[1 + 6 * NBUF:]
    c = lax.axis_index("c")
    s = lax.axis_index("s")
    wid = c * NS + s
    rbase = pl.multiple_of(s * rps, 8)

    pltpu.sync_copy(zr_hbm.at[pl.ds(rbase, rps)], acc.at[pl.ds(rbase, rps)])
    if with_cnt:
      pltpu.sync_copy(zc_hbm.at[pl.ds(rbase, rps)], cacc.at[pl.ds(rbase, rps)])

      @pl.loop(0, CHUNK, step=16)
      def _(k):
        ones[pl.ds(k, 16)] = jnp.ones((16,), jnp.float32)

    plsc.subcore_barrier()

    base = wid * per_worker

    def load_and_gather(k, chunk):
      off = pl.multiple_of(base + chunk * CHUNK, 8)
      pltpu.sync_copy(row_hbm.at[pl.ds(off, CHUNK)], ridx[k])
      pltpu.sync_copy(col_hbm.at[pl.ds(off, CHUNK)], cidx[k])
      pltpu.async_copy(h_hbm.at[ridx[k]], rows[k], gsem[k])

    def wait_gather(k):
      pltpu.make_async_copy(h_hbm.at[ridx[k]], rows[k], gsem[k]).wait()

    def start_scatter(k):
      sc = pltpu.async_copy(rows[k], acc.at[cidx[k]], ssem[k], add=True)
      oc = None
      if with_cnt:
        oc = pltpu.async_copy(ones, cacc.at[cidx[k]], osem[k], add=True)
      return sc, oc

    for k in range(NBUF):
      load_and_gather(k, k)

    @pl.loop(0, n_chunks - NBUF, step=NBUF)
    def _(i):
      scs = []
      for k in range(NBUF):
        wait_gather(k)
        scs.append(start_scatter(k))
      for k in range(NBUF):
        sc, oc = scs[k]
        sc.wait()
        if oc is not None:
          oc.wait()
        load_and_gather(k, i + NBUF + k)

    scs = []
    for k in range(NBUF):
      wait_gather(k)
      scs.append(start_scatter(k))
    for sc, oc in scs:
      sc.wait()
      if oc is not None:
        oc.wait()

    plsc.subcore_barrier()

    pltpu.sync_copy(acc.at[pl.ds(rbase, rps)],
                    p_hbm.at[c].at[pl.ds(rbase, rps)])
    if with_cnt:
      pltpu.sync_copy(cacc.at[pl.ds(rbase, rps)],
                      cnt_hbm.at[c].at[pl.ds(rbase, rps)])

  return pl.kernel(body, out_type=tuple(out_type), mesh=mesh,
                   scratch_types=scratch)


def _dense_body(p0, p1, c0, c1, w1t, b1r, w2t, out):
  s = p0[...] + p1[...]
  cc = c0[...] + c1[...]
  inv = jnp.where(cc > 0.0, 1.0 / cc, 0.0)
  h = jnp.dot(s * inv, w1t[...], preferred_element_type=jnp.float32)
  h = jnp.maximum(h + b1r[...], 0.0)
  out[...] = jnp.dot(h, w2t[...], preferred_element_type=jnp.float32)


def _finish_body(p0, p1, c0, c1, b2r, out):
  s = p0[...] + p1[...]
  cc = c0[...] + c1[...]
  inv = jnp.where(cc > 0.0, 1.0 / cc, 0.0)
  r = s * inv + b2r[...]
  nrm = jnp.sqrt(jnp.sum(r * r, axis=1, keepdims=True))
  out[...] = r / jnp.maximum(nrm, 1e-12)


def _row_specs():
  return [
      pl.BlockSpec((BM, D), lambda i: (i, 0)),
      pl.BlockSpec((BM, D), lambda i: (i, 0)),
      pl.BlockSpec((BM, 1), lambda i: (i, 0)),
      pl.BlockSpec((BM, 1), lambda i: (i, 0)),
  ]


@jax.jit
def kernel(x, edge_index, W1, b1, W2, b2):
  e = edge_index.shape[1]
  gran = NW * CHUNK * NBUF
  e_pad = max(-(-e // gran), 2) * gran
  row = edge_index[0]
  col = edge_index[1]
  if e_pad != e:
    row = jnp.concatenate([row, jnp.zeros((e_pad - e,), jnp.int32)])
    col = jnp.concatenate([col, jnp.full((e_pad - e,), N, jnp.int32)])
  zr = jnp.zeros((NPAD, D), jnp.float32)
  zc = jnp.zeros((NPAD,), jnp.float32)

  p1, cnt = _make_spmm(e_pad, True)(x, row, col, zr, zc)
  c0 = cnt[0][:, None]
  c1 = cnt[1][:, None]

  grid = (NPAD // BM,)
  b = pl.pallas_call(
      _dense_body,
      grid=grid,
      in_specs=_row_specs() + [
          pl.BlockSpec((D, D_HID), lambda i: (0, 0)),
          pl.BlockSpec((1, D_HID), lambda i: (0, 0)),
          pl.BlockSpec((D_HID, D), lambda i: (0, 0)),
      ],
      out_specs=pl.BlockSpec((BM, D), lambda i: (i, 0)),
      out_shape=jax.ShapeDtypeStruct((NPAD, D), jnp.float32),
  )(p1[0], p1[1], c0, c1, W1.T, b1[None, :], W2.T)

  (p2,) = _make_spmm(e_pad, False)(b, row, col, zr)

  out = pl.pallas_call(
      _finish_body,
      grid=grid,
      in_specs=_row_specs() + [pl.BlockSpec((1, D), lambda i: (0, 0))],
      out_specs=pl.BlockSpec((BM, D), lambda i: (i, 0)),
      out_shape=jax.ShapeDtypeStruct((NPAD, D), jnp.float32),
  )(p2[0], p2[1], c0, c1, b2[None, :])
  return out[:N]


# speedup vs baseline: 1.5847x; 1.5847x over previous
import functools

import jax
import jax.numpy as jnp
from jax import lax
from jax.experimental import pallas as pl
from jax.experimental.pallas import tpu as pltpu
from jax.experimental.pallas import tpu_sc as plsc

N = 10000
D = 128
D_HID = 256
NPAD = 10240
NC = 2
NS = 16
NW = NC * NS
CHUNK = 128
FRAC0 = 0.63
BM = 512


def _make_spmm(e_pad, with_cnt):
  n_total = e_pad // CHUNK
  assert n_total % NS == 0
  n0 = int(round(n_total // NS * FRAC0))
  n1 = n_total // NS - n0
  rps = NPAD // NS

  mesh = plsc.VectorSubcoreMesh(core_axis_name="c", subcore_axis_name="s")
  out_type = [jax.ShapeDtypeStruct((NC, NPAD, D), jnp.float32)]
  scratch = [
      pltpu.VMEM_SHARED((NPAD, D), jnp.float32),
      pltpu.VMEM((CHUNK,), jnp.int32),
      pltpu.VMEM((CHUNK,), jnp.int32),
      pltpu.VMEM((CHUNK, D), jnp.float32),
      pltpu.SemaphoreType.DMA,
  ]
  if with_cnt:
    out_type.append(jax.ShapeDtypeStruct((NC, NPAD), jnp.float32))
    scratch.append(pltpu.VMEM_SHARED((NPAD,), jnp.float32))
    scratch.append(pltpu.VMEM((CHUNK,), jnp.float32))

  def body(*refs):
    if with_cnt:
      (h_hbm, row_hbm, col_hbm, zr_hbm, zc_hbm, p_hbm, cnt_hbm,
       acc, ridx, cidx, rows, sem, cacc, ones) = refs
    else:
      (h_hbm, row_hbm, col_hbm, zr_hbm, p_hbm,
       acc, ridx, cidx, rows, sem) = refs
    c = lax.axis_index("c")
    s = lax.axis_index("s")
    rbase = pl.multiple_of(s * rps, 8)

    pltpu.sync_copy(zr_hbm.at[pl.ds(rbase, rps)], acc.at[pl.ds(rbase, rps)])
    if with_cnt:
      pltpu.sync_copy(zc_hbm.at[pl.ds(rbase, rps)], cacc.at[pl.ds(rbase, rps)])

      @pl.loop(0, CHUNK, step=16)
      def _(k):
        ones[pl.ds(k, 16)] = jnp.ones((16,), jnp.float32)

    plsc.subcore_barrier()

    base = jnp.where(c == 0, s * n0, NS * n0 + s * n1) * CHUNK
    nloc = jnp.where(c == 0, n0, n1)

    @pl.loop(0, max(n0, n1))
    def _(i):
      @pl.when(i < nloc)
      def _():
        off = pl.multiple_of(base + i * CHUNK, 8)
        pltpu.sync_copy(row_hbm.at[pl.ds(off, CHUNK)], ridx)
        pltpu.sync_copy(col_hbm.at[pl.ds(off, CHUNK)], cidx)
        pltpu.async_copy(h_hbm.at[ridx], rows, sem).wait()
        pltpu.sync_copy(rows, acc.at[cidx], add=True)
        if with_cnt:
          pltpu.sync_copy(ones, cacc.at[cidx], add=True)

    plsc.subcore_barrier()

    pltpu.sync_copy(acc.at[pl.ds(rbase, rps)],
                    p_hbm.at[c].at[pl.ds(rbase, rps)])
    if with_cnt:
      pltpu.sync_copy(cacc.at[pl.ds(rbase, rps)],
                      cnt_hbm.at[c].at[pl.ds(rbase, rps)])

  return pl.kernel(body, out_type=tuple(out_type), mesh=mesh,
                   scratch_types=scratch)


def _dense_body(p0, p1, c0, c1, w1t, b1r, w2t, out):
  s = p0[...] + p1[...]
  cc = c0[...] + c1[...]
  inv = jnp.where(cc > 0.0, 1.0 / cc, 0.0)
  h = jnp.dot(s * inv, w1t[...], preferred_element_type=jnp.float32)
  h = jnp.maximum(h + b1r[...], 0.0)
  out[...] = jnp.dot(h, w2t[...], preferred_element_type=jnp.float32)


def _finish_body(p0, p1, c0, c1, b2r, out):
  s = p0[...] + p1[...]
  cc = c0[...] + c1[...]
  inv = jnp.where(cc > 0.0, 1.0 / cc, 0.0)
  r = s * inv + b2r[...]
  nrm = jnp.sqrt(jnp.sum(r * r, axis=1, keepdims=True))
  out[...] = r / jnp.maximum(nrm, 1e-12)


def _row_specs():
  return [
      pl.BlockSpec((BM, D), lambda i: (i, 0)),
      pl.BlockSpec((BM, D), lambda i: (i, 0)),
      pl.BlockSpec((BM, 1), lambda i: (i, 0)),
      pl.BlockSpec((BM, 1), lambda i: (i, 0)),
  ]


@jax.jit
def kernel(x, edge_index, W1, b1, W2, b2):
  e = edge_index.shape[1]
  gran = NS * CHUNK
  e_pad = -(-e // gran) * gran
  row = edge_index[0]
  col = edge_index[1]
  if e_pad != e:
    row = jnp.concatenate([row, jnp.zeros((e_pad - e,), jnp.int32)])
    col = jnp.concatenate([col, jnp.full((e_pad - e,), N, jnp.int32)])
  zr = jnp.zeros((NPAD, D), jnp.float32)
  zc = jnp.zeros((NPAD,), jnp.float32)

  p1, cnt = _make_spmm(e_pad, True)(x, row, col, zr, zc)
  c0 = cnt[0][:, None]
  c1 = cnt[1][:, None]

  grid = (NPAD // BM,)
  b = pl.pallas_call(
      _dense_body,
      grid=grid,
      in_specs=_row_specs() + [
          pl.BlockSpec((D, D_HID), lambda i: (0, 0)),
          pl.BlockSpec((1, D_HID), lambda i: (0, 0)),
          pl.BlockSpec((D_HID, D), lambda i: (0, 0)),
      ],
      out_specs=pl.BlockSpec((BM, D), lambda i: (i, 0)),
      out_shape=jax.ShapeDtypeStruct((NPAD, D), jnp.float32),
  )(p1[0], p1[1], c0, c1, W1.T, b1[None, :], W2.T)

  (p2,) = _make_spmm(e_pad, False)(b, row, col, zr)

  out = pl.pallas_call(
      _finish_body,
      grid=grid,
      in_specs=_row_specs() + [pl.BlockSpec((1, D), lambda i: (0, 0))],
      out_specs=pl.BlockSpec((BM, D), lambda i: (i, 0)),
      out_shape=jax.ShapeDtypeStruct((NPAD, D), jnp.float32),
  )(p2[0], p2[1], c0, c1, b2[None, :])
  return out[:N]


# speedup vs baseline: 1.7869x; 1.1276x over previous
import jax
import jax.numpy as jnp
from jax import lax
from jax.experimental import pallas as pl
from jax.experimental.pallas import tpu as pltpu
from jax.experimental.pallas import tpu_sc as plsc

N = 10000
D = 128
D_HID = 256
NPAD = 10240
NC = 2
NS = 16
CHUNK = 128
FRAC0 = 0.59
BM = 2000


def _make_spmm(n_rows, e_pad, with_cnt):
  n_total = e_pad // CHUNK
  c0_total = int(round(n_total * FRAC0))
  n0, r0 = divmod(c0_total, NS)
  n1, r1 = divmod(n_total - c0_total, NS)
  n_max = max(n0 + (r0 > 0), n1 + (r1 > 0))
  rps = NPAD // NS

  mesh = plsc.VectorSubcoreMesh(core_axis_name="c", subcore_axis_name="s")
  out_type = [jax.ShapeDtypeStruct((NC, NPAD, D), jnp.float32)]
  scratch = [
      pltpu.VMEM_SHARED((NPAD, D), jnp.float32),
      pltpu.VMEM((CHUNK,), jnp.int32),
      pltpu.VMEM((CHUNK,), jnp.int32),
      pltpu.VMEM((CHUNK, D), jnp.float32),
      pltpu.SemaphoreType.DMA,
  ]
  if with_cnt:
    out_type.append(jax.ShapeDtypeStruct((NC, NPAD), jnp.float32))
    scratch.append(pltpu.VMEM_SHARED((NPAD,), jnp.float32))
    scratch.append(pltpu.VMEM((CHUNK,), jnp.float32))

  def body(*refs):
    if with_cnt:
      (h_hbm, ei_hbm, zr_hbm, zc_hbm, p_hbm, cnt_hbm,
       acc, ridx, cidx, rows, sem, cacc, ones) = refs
    else:
      (h_hbm, ei_hbm, zr_hbm, p_hbm,
       acc, ridx, cidx, rows, sem) = refs
    c = lax.axis_index("c")
    s = lax.axis_index("s")
    rbase = pl.multiple_of(s * rps, 8)

    pltpu.sync_copy(zr_hbm, acc.at[pl.ds(rbase, rps)])
    if with_cnt:
      pltpu.sync_copy(zc_hbm, cacc.at[pl.ds(rbase, rps)])

      @pl.loop(0, CHUNK, step=16)
      def _(k):
        ones[pl.ds(k, 16)] = jnp.ones((16,), jnp.float32)

    plsc.subcore_barrier()

    start0 = s * n0 + jnp.minimum(s, r0)
    count0 = n0 + (s < r0).astype(jnp.int32)
    start1 = c0_total + s * n1 + jnp.minimum(s, r1)
    count1 = n1 + (s < r1).astype(jnp.int32)
    base = jnp.where(c == 0, start0, start1) * CHUNK
    nloc = jnp.where(c == 0, count0, count1)

    @pl.loop(0, n_max)
    def _(i):
      @pl.when(i < nloc)
      def _():
        off = pl.multiple_of(base + i * CHUNK, 8)
        pltpu.sync_copy(ei_hbm.at[0].at[pl.ds(off, CHUNK)], ridx)
        pltpu.sync_copy(ei_hbm.at[1].at[pl.ds(off, CHUNK)], cidx)
        pltpu.async_copy(h_hbm.at[ridx], rows, sem).wait()
        pltpu.sync_copy(rows, acc.at[cidx], add=True)
        if with_cnt:
          pltpu.sync_copy(ones, cacc.at[cidx], add=True)

    plsc.subcore_barrier()

    pltpu.sync_copy(acc.at[pl.ds(rbase, rps)],
                    p_hbm.at[c].at[pl.ds(rbase, rps)])
    if with_cnt:
      pltpu.sync_copy(cacc.at[pl.ds(rbase, rps)],
                      cnt_hbm.at[c].at[pl.ds(rbase, rps)])

  return pl.kernel(body, out_type=tuple(out_type), mesh=mesh,
                   scratch_types=scratch)


def _dense_body(p0, p1, c0, c1, w1, b1r, w2, out):
  s = p0[...] + p1[...]
  cc = c0[...] + c1[...]
  inv = jnp.where(cc > 0.0, 1.0 / cc, 0.0)
  h = lax.dot_general(s * inv, w1[...], (((1,), (1,)), ((), ())),
                      preferred_element_type=jnp.float32)
  h = jnp.maximum(h + b1r[...], 0.0)
  out[...] = lax.dot_general(h, w2[...], (((1,), (1,)), ((), ())),
                             preferred_element_type=jnp.float32)


def _finish_body(p0, p1, c0, c1, b2r, out):
  s = p0[...] + p1[...]
  cc = c0[...] + c1[...]
  inv = jnp.where(cc > 0.0, 1.0 / cc, 0.0)
  r = s * inv + b2r[...]
  nrm = jnp.sqrt(jnp.sum(r * r, axis=1, keepdims=True))
  out[...] = r / jnp.maximum(nrm, 1e-12)


def _row_specs():
  return [
      pl.BlockSpec((BM, D), lambda i: (i, 0)),
      pl.BlockSpec((BM, D), lambda i: (i, 0)),
      pl.BlockSpec((BM, 1), lambda i: (i, 0)),
      pl.BlockSpec((BM, 1), lambda i: (i, 0)),
  ]


@jax.jit
def kernel(x, edge_index, W1, b1, W2, b2):
  e = edge_index.shape[1]
  e_pad = -(-e // CHUNK) * CHUNK
  if e_pad != e:
    pad = jnp.concatenate(
        [jnp.zeros((1, e_pad - e), jnp.int32),
         jnp.full((1, e_pad - e), N, jnp.int32)])
    edge_index = jnp.concatenate([edge_index, pad], axis=1)
  zr = jnp.zeros((NPAD // NS, D), jnp.float32)
  zc = jnp.zeros((NPAD // NS,), jnp.float32)

  p1, cnt = _make_spmm(N, e_pad, True)(x, edge_index, zr, zc)
  c0 = cnt[0][:, None]
  c1 = cnt[1][:, None]

  grid = (N // BM,)
  b = pl.pallas_call(
      _dense_body,
      grid=grid,
      in_specs=_row_specs() + [
          pl.BlockSpec((D_HID, D), lambda i: (0, 0)),
          pl.BlockSpec((1, D_HID), lambda i: (0, 0)),
          pl.BlockSpec((D, D_HID), lambda i: (0, 0)),
      ],
      out_specs=pl.BlockSpec((BM, D), lambda i: (i, 0)),
      out_shape=jax.ShapeDtypeStruct((N, D), jnp.float32),
  )(p1[0], p1[1], c0, c1, W1, b1[None, :], W2)

  (p2,) = _make_spmm(N, e_pad, False)(b, edge_index, zr)

  out = pl.pallas_call(
      _finish_body,
      grid=grid,
      in_specs=_row_specs() + [pl.BlockSpec((1, D), lambda i: (0, 0))],
      out_specs=pl.BlockSpec((BM, D), lambda i: (i, 0)),
      out_shape=jax.ShapeDtypeStruct((N, D), jnp.float32),
  )(p2[0], p2[1], c0, c1, b2[None, :])
  return out


# speedup vs baseline: 2.3206x; 1.2987x over previous
import jax
import jax.numpy as jnp
from jax import lax
from jax.experimental import pallas as pl
from jax.experimental.pallas import tpu as pltpu
from jax.experimental.pallas import tpu_sc as plsc

N = 10000
D = 128
D_HID = 256
NPAD = 10240
NC = 2
NS = 16
CHUNK = 128
FRAC0 = 0.52
BM = 2000


def _make_spmm(n_rows, e_pad, with_cnt):
  n_total = e_pad // CHUNK
  c0_total = int(round(n_total * FRAC0))
  n0, r0 = divmod(c0_total, NS)
  n1, r1 = divmod(n_total - c0_total, NS)
  n_max = max(n0 + (r0 > 0), n1 + (r1 > 0))
  rps = NPAD // NS

  mesh = plsc.VectorSubcoreMesh(core_axis_name="c", subcore_axis_name="s")
  out_type = [jax.ShapeDtypeStruct((NPAD, D), jnp.float32),
              jax.ShapeDtypeStruct((NPAD, D), jnp.float32)]
  scratch = [
      pltpu.VMEM_SHARED((NPAD, D), jnp.float32),
      pltpu.VMEM((2, CHUNK), jnp.int32),
      pltpu.VMEM((CHUNK, D), jnp.float32),
      pltpu.SemaphoreType.DMA,
  ]
  if with_cnt:
    out_type.append(jax.ShapeDtypeStruct((NC, NPAD), jnp.float32))
    scratch.append(pltpu.VMEM_SHARED((NPAD,), jnp.float32))
    scratch.append(pltpu.VMEM((CHUNK,), jnp.float32))

  def body(*refs):
    if with_cnt:
      (h_hbm, ei_hbm, zr_hbm, zc_hbm, pa_hbm, pb_hbm, cnt_hbm,
       acc, eidx, rows, sem, cacc, ones) = refs
    else:
      (h_hbm, ei_hbm, zr_hbm, pa_hbm, pb_hbm,
       acc, eidx, rows, sem) = refs
    c = lax.axis_index("c")
    s = lax.axis_index("s")
    rbase = pl.multiple_of(s * rps, 8)

    pltpu.sync_copy(zr_hbm, acc.at[pl.ds(rbase, rps)])
    if with_cnt:
      pltpu.sync_copy(zc_hbm, cacc.at[pl.ds(rbase, rps)])

      @pl.loop(0, CHUNK, step=16)
      def _(k):
        ones[pl.ds(k, 16)] = jnp.ones((16,), jnp.float32)

    plsc.subcore_barrier()

    start0 = s * n0 + jnp.minimum(s, r0)
    count0 = n0 + (s < r0).astype(jnp.int32)
    start1 = c0_total + s * n1 + jnp.minimum(s, r1)
    count1 = n1 + (s < r1).astype(jnp.int32)
    base = jnp.where(c == 0, start0, start1) * CHUNK
    nloc = jnp.where(c == 0, count0, count1)

    @pl.loop(0, n_max)
    def _(i):
      @pl.when(i < nloc)
      def _():
        off = pl.multiple_of(base + i * CHUNK, 8)
        pltpu.sync_copy(ei_hbm.at[:, pl.ds(off, CHUNK)], eidx)
        pltpu.async_copy(h_hbm.at[eidx.at[0]], rows, sem).wait()
        pltpu.sync_copy(rows, acc.at[eidx.at[1]], add=True)
        if with_cnt:
          pltpu.sync_copy(ones, cacc.at[eidx.at[1]], add=True)

    plsc.subcore_barrier()

    @pl.when(c == 0)
    def _():
      pltpu.sync_copy(acc.at[pl.ds(rbase, rps)], pa_hbm.at[pl.ds(rbase, rps)])

    @pl.when(c == 1)
    def _():
      pltpu.sync_copy(acc.at[pl.ds(rbase, rps)], pb_hbm.at[pl.ds(rbase, rps)])

    if with_cnt:
      pltpu.sync_copy(cacc.at[pl.ds(rbase, rps)],
                      cnt_hbm.at[c].at[pl.ds(rbase, rps)])

  return pl.kernel(body, out_type=tuple(out_type), mesh=mesh,
                   scratch_types=scratch)


def _dense_body(p0, p1, c0, c1, w1, b1r, w2, out):
  s = p0[...] + p1[...]
  cc = c0[...] + c1[...]
  inv = jnp.where(cc > 0.0, 1.0 / cc, 0.0)
  h = lax.dot_general(s * inv, w1[...], (((1,), (1,)), ((), ())),
                      preferred_element_type=jnp.float32)
  h = jnp.maximum(h + b1r[...], 0.0)
  out[...] = lax.dot_general(h, w2[...], (((1,), (1,)), ((), ())),
                             preferred_element_type=jnp.float32)


def _finish_body(p0, p1, c0, c1, b2r, out):
  s = p0[...] + p1[...]
  cc = c0[...] + c1[...]
  inv = jnp.where(cc > 0.0, 1.0 / cc, 0.0)
  r = s * inv + b2r[...]
  nrm = jnp.sqrt(jnp.sum(r * r, axis=1, keepdims=True))
  out[...] = r / jnp.maximum(nrm, 1e-12)


def _row_specs():
  return [
      pl.BlockSpec((BM, D), lambda i: (i, 0)),
      pl.BlockSpec((BM, D), lambda i: (i, 0)),
      pl.BlockSpec((BM, 1), lambda i: (i, 0)),
      pl.BlockSpec((BM, 1), lambda i: (i, 0)),
  ]


@jax.jit
def kernel(x, edge_index, W1, b1, W2, b2):
  e = edge_index.shape[1]
  e_pad = -(-e // CHUNK) * CHUNK
  if e_pad != e:
    pad = jnp.concatenate(
        [jnp.zeros((1, e_pad - e), jnp.int32),
         jnp.full((1, e_pad - e), N, jnp.int32)])
    edge_index = jnp.concatenate([edge_index, pad], axis=1)
  zr = jnp.zeros((NPAD // NS, D), jnp.float32)
  zc = jnp.zeros((NPAD // NS,), jnp.float32)

  p1a, p1b, cnt = _make_spmm(N, e_pad, True)(x, edge_index, zr, zc)
  c0 = cnt[0][:, None]
  c1 = cnt[1][:, None]

  grid = (N // BM,)
  b = pl.pallas_call(
      _dense_body,
      grid=grid,
      in_specs=_row_specs() + [
          pl.BlockSpec((D_HID, D), lambda i: (0, 0)),
          pl.BlockSpec((1, D_HID), lambda i: (0, 0)),
          pl.BlockSpec((D, D_HID), lambda i: (0, 0)),
      ],
      out_specs=pl.BlockSpec((BM, D), lambda i: (i, 0)),
      out_shape=jax.ShapeDtypeStruct((N, D), jnp.float32),
  )(p1a, p1b, c0, c1, W1, b1[None, :], W2)

  p2a, p2b = _make_spmm(N, e_pad, False)(b, edge_index, zr)

  out = pl.pallas_call(
      _finish_body,
      grid=grid,
      in_specs=_row_specs() + [pl.BlockSpec((1, D), lambda i: (0, 0))],
      out_specs=pl.BlockSpec((BM, D), lambda i: (i, 0)),
      out_shape=jax.ShapeDtypeStruct((N, D), jnp.float32),
  )(p2a, p2b, c0, c1, b2[None, :])
  return out


# speedup vs baseline: 3.7619x; 1.6211x over previous
import jax
import jax.numpy as jnp
from jax import lax
from jax.experimental import pallas as pl
from jax.experimental.pallas import tpu as pltpu
from jax.experimental.pallas import tpu_sc as plsc

N = 10000
D = 128
D_HID = 256
NPAD = 10240
NC = 2
NS = 16
CHUNK = 128
FRAC0 = 0.52
BM = 2000


def _make_spmm(n_rows, e_pad, with_cnt):
  n_total = e_pad // CHUNK
  c0_total = int(round(n_total * FRAC0))
  n0, r0 = divmod(c0_total, NS)
  n1, r1 = divmod(n_total - c0_total, NS)
  n_max = max(n0 + (r0 > 0), n1 + (r1 > 0))
  rps = NPAD // NS

  mesh = plsc.VectorSubcoreMesh(core_axis_name="c", subcore_axis_name="s")
  out_type = [jax.ShapeDtypeStruct((NPAD, D), jnp.float32),
              jax.ShapeDtypeStruct((NPAD, D), jnp.float32)]
  scratch = [
      pltpu.VMEM_SHARED((NPAD, D), jnp.float32),
      pltpu.VMEM((2, CHUNK), jnp.int32),
      pltpu.VMEM((2, CHUNK), jnp.int32),
      pltpu.VMEM((CHUNK, D), jnp.float32),
      pltpu.VMEM((CHUNK, D), jnp.float32),
      pltpu.SemaphoreType.DMA,
      pltpu.SemaphoreType.DMA,
  ]
  if with_cnt:
    out_type.append(jax.ShapeDtypeStruct((NC, NPAD), jnp.float32))
    scratch.append(pltpu.VMEM_SHARED((NPAD,), jnp.float32))
    scratch.append(pltpu.VMEM((CHUNK,), jnp.float32))

  def body(*refs):
    if with_cnt:
      (h_hbm, ei_hbm, zr_hbm, zc_hbm, pa_hbm, pb_hbm, cnt_hbm,
       acc, eidxa, eidxb, rowsa, rowsb, sema, semb, cacc, ones) = refs
    else:
      (h_hbm, ei_hbm, zr_hbm, pa_hbm, pb_hbm,
       acc, eidxa, eidxb, rowsa, rowsb, sema, semb) = refs
    c = lax.axis_index("c")
    s = lax.axis_index("s")
    rbase = pl.multiple_of(s * rps, 8)

    pltpu.sync_copy(zr_hbm, acc.at[pl.ds(rbase, rps)])
    if with_cnt:
      pltpu.sync_copy(zc_hbm, cacc.at[pl.ds(rbase, rps)])

      @pl.loop(0, CHUNK, step=16)
      def _(k):
        ones[pl.ds(k, 16)] = jnp.ones((16,), jnp.float32)

    plsc.subcore_barrier()

    start0 = s * n0 + jnp.minimum(s, r0)
    count0 = n0 + (s < r0).astype(jnp.int32)
    start1 = c0_total + s * n1 + jnp.minimum(s, r1)
    count1 = n1 + (s < r1).astype(jnp.int32)
    base = jnp.where(c == 0, start0, start1) * CHUNK
    nloc = jnp.where(c == 0, count0, count1)

    def load_and_gather(eidx, rows, sem, chunk):
      off = pl.multiple_of(base + chunk * CHUNK, 8)
      pltpu.sync_copy(ei_hbm.at[:, pl.ds(off, CHUNK)], eidx)
      pltpu.async_copy(h_hbm.at[eidx.at[0]], rows, sem)

    def wait_gather(eidx, rows, sem):
      pltpu.make_async_copy(h_hbm.at[eidx.at[0]], rows, sem).wait()

    def scatter(eidx, rows):
      pltpu.sync_copy(rows, acc.at[eidx.at[1]], add=True)
      if with_cnt:
        pltpu.sync_copy(ones, cacc.at[eidx.at[1]], add=True)

    @pl.when(nloc > 0)
    def _():
      load_and_gather(eidxa, rowsa, sema, 0)

    @pl.loop(0, n_max, step=2)
    def _(i):
      @pl.when(i < nloc)
      def _():
        @pl.when(i + 1 < nloc)
        def _():
          load_and_gather(eidxb, rowsb, semb, i + 1)
        wait_gather(eidxa, rowsa, sema)
        scatter(eidxa, rowsa)

        @pl.when(i + 1 < nloc)
        def _():
          @pl.when(i + 2 < nloc)
          def _():
            load_and_gather(eidxa, rowsa, sema, i + 2)
          wait_gather(eidxb, rowsb, semb)
          scatter(eidxb, rowsb)

    plsc.subcore_barrier()

    @pl.when(c == 0)
    def _():
      pltpu.sync_copy(acc.at[pl.ds(rbase, rps)], pa_hbm.at[pl.ds(rbase, rps)])

    @pl.when(c == 1)
    def _():
      pltpu.sync_copy(acc.at[pl.ds(rbase, rps)], pb_hbm.at[pl.ds(rbase, rps)])

    if with_cnt:
      pltpu.sync_copy(cacc.at[pl.ds(rbase, rps)],
                      cnt_hbm.at[c].at[pl.ds(rbase, rps)])

  return pl.kernel(body, out_type=tuple(out_type), mesh=mesh,
                   scratch_types=scratch)


def _dense_body(p0, p1, c0, c1, w1, b1r, w2, out):
  s = p0[...] + p1[...]
  cc = c0[...] + c1[...]
  inv = jnp.where(cc > 0.0, 1.0 / cc, 0.0)
  h = lax.dot_general(s * inv, w1[...], (((1,), (1,)), ((), ())),
                      preferred_element_type=jnp.float32)
  h = jnp.maximum(h + b1r[...], 0.0)
  out[...] = lax.dot_general(h, w2[...], (((1,), (1,)), ((), ())),
                             preferred_element_type=jnp.float32)


def _finish_body(p0, p1, c0, c1, b2r, out):
  s = p0[...] + p1[...]
  cc = c0[...] + c1[...]
  inv = jnp.where(cc > 0.0, 1.0 / cc, 0.0)
  r = s * inv + b2r[...]
  nrm = jnp.sqrt(jnp.sum(r * r, axis=1, keepdims=True))
  out[...] = r / jnp.maximum(nrm, 1e-12)


def _row_specs():
  return [
      pl.BlockSpec((BM, D), lambda i: (i, 0)),
      pl.BlockSpec((BM, D), lambda i: (i, 0)),
      pl.BlockSpec((BM, 1), lambda i: (i, 0)),
      pl.BlockSpec((BM, 1), lambda i: (i, 0)),
  ]


@jax.jit
def kernel(x, edge_index, W1, b1, W2, b2):
  e = edge_index.shape[1]
  e_pad = -(-e // CHUNK) * CHUNK
  if e_pad != e:
    pad = jnp.concatenate(
        [jnp.zeros((1, e_pad - e), jnp.int32),
         jnp.full((1, e_pad - e), N, jnp.int32)])
    edge_index = jnp.concatenate([edge_index, pad], axis=1)
  zr = jnp.zeros((NPAD // NS, D), jnp.float32)
  zc = jnp.zeros((NPAD // NS,), jnp.float32)

  p1a, p1b, cnt = _make_spmm(N, e_pad, True)(x, edge_index, zr, zc)
  c0 = cnt[0][:, None]
  c1 = cnt[1][:, None]

  grid = (N // BM,)
  b = pl.pallas_call(
      _dense_body,
      grid=grid,
      in_specs=_row_specs() + [
          pl.BlockSpec((D_HID, D), lambda i: (0, 0)),
          pl.BlockSpec((1, D_HID), lambda i: (0, 0)),
          pl.BlockSpec((D, D_HID), lambda i: (0, 0)),
      ],
      out_specs=pl.BlockSpec((BM, D), lambda i: (i, 0)),
      out_shape=jax.ShapeDtypeStruct((N, D), jnp.float32),
  )(p1a, p1b, c0, c1, W1, b1[None, :], W2)

  p2a, p2b = _make_spmm(N, e_pad, False)(b, edge_index, zr)

  out = pl.pallas_call(
      _finish_body,
      grid=grid,
      in_specs=_row_specs() + [pl.BlockSpec((1, D), lambda i: (0, 0))],
      out_specs=pl.BlockSpec((BM, D), lambda i: (i, 0)),
      out_shape=jax.ShapeDtypeStruct((N, D), jnp.float32),
  )(p2a, p2b, c0, c1, b2[None, :])
  return out


# speedup vs baseline: 3.8528x; 1.0242x over previous
import jax
import jax.numpy as jnp
from jax import lax
from jax.experimental import pallas as pl
from jax.experimental.pallas import tpu as pltpu
from jax.experimental.pallas import tpu_sc as plsc

N = 10000
D = 128
D_HID = 256
NPAD = 10240
NC = 2
NS = 16
CHUNK = 128
FRAC0 = 0.50
BM = 2000


def _make_spmm(n_rows, e_pad, with_cnt):
  n_total = e_pad // CHUNK
  c0_total = int(round(n_total * FRAC0))
  n0, r0 = divmod(c0_total, NS)
  n1, r1 = divmod(n_total - c0_total, NS)
  n_max = max(n0 + (r0 > 0), n1 + (r1 > 0))
  rps = NPAD // NS

  mesh = plsc.VectorSubcoreMesh(core_axis_name="c", subcore_axis_name="s")
  out_type = [jax.ShapeDtypeStruct((NPAD, D), jnp.float32),
              jax.ShapeDtypeStruct((NPAD, D), jnp.float32)]
  scratch = [
      pltpu.VMEM_SHARED((NPAD, D), jnp.float32),
      pltpu.VMEM((2, CHUNK), jnp.int32),
      pltpu.VMEM((2, CHUNK), jnp.int32),
      pltpu.VMEM((CHUNK, D), jnp.float32),
      pltpu.VMEM((CHUNK, D), jnp.float32),
      pltpu.SemaphoreType.DMA,
      pltpu.SemaphoreType.DMA,
  ]
  if with_cnt:
    out_type.append(jax.ShapeDtypeStruct((NC, NPAD), jnp.float32))
    scratch.append(pltpu.VMEM_SHARED((NPAD,), jnp.float32))
    scratch.append(pltpu.VMEM((CHUNK,), jnp.float32))

  def body(*refs):
    if with_cnt:
      (h_hbm, ei_hbm, zr_hbm, zc_hbm, pa_hbm, pb_hbm, cnt_hbm,
       acc, eidxa, eidxb, rowsa, rowsb, sema, semb, cacc, ones) = refs
    else:
      (h_hbm, ei_hbm, zr_hbm, pa_hbm, pb_hbm,
       acc, eidxa, eidxb, rowsa, rowsb, sema, semb) = refs
    c = lax.axis_index("c")
    s = lax.axis_index("s")
    rbase = pl.multiple_of(s * rps, 8)

    pltpu.sync_copy(zr_hbm, acc.at[pl.ds(rbase, rps)])
    if with_cnt:
      pltpu.sync_copy(zc_hbm, cacc.at[pl.ds(rbase, rps)])

      @pl.loop(0, CHUNK, step=16)
      def _(k):
        ones[pl.ds(k, 16)] = jnp.ones((16,), jnp.float32)

    plsc.subcore_barrier()

    start0 = s * n0 + jnp.minimum(s, r0)
    count0 = n0 + (s < r0).astype(jnp.int32)
    start1 = c0_total + s * n1 + jnp.minimum(s, r1)
    count1 = n1 + (s < r1).astype(jnp.int32)
    base = jnp.where(c == 0, start0, start1) * CHUNK
    nloc = jnp.where(c == 0, count0, count1)

    def load_and_gather(eidx, rows, sem, chunk):
      off = pl.multiple_of(base + chunk * CHUNK, 8)
      pltpu.sync_copy(ei_hbm.at[:, pl.ds(off, CHUNK)], eidx)
      pltpu.async_copy(h_hbm.at[eidx.at[0]], rows, sem)

    def wait_gather(eidx, rows, sem):
      pltpu.make_async_copy(h_hbm.at[eidx.at[0]], rows, sem).wait()

    def scatter(eidx, rows):
      pltpu.sync_copy(rows, acc.at[eidx.at[1]], add=True)
      if with_cnt:
        pltpu.sync_copy(ones, cacc.at[eidx.at[1]], add=True)

    @pl.when(nloc > 0)
    def _():
      load_and_gather(eidxa, rowsa, sema, 0)

    @pl.loop(0, n_max, step=2)
    def _(i):
      @pl.when(i < nloc)
      def _():
        @pl.when(i + 1 < nloc)
        def _():
          load_and_gather(eidxb, rowsb, semb, i + 1)
        wait_gather(eidxa, rowsa, sema)
        scatter(eidxa, rowsa)

        @pl.when(i + 1 < nloc)
        def _():
          @pl.when(i + 2 < nloc)
          def _():
            load_and_gather(eidxa, rowsa, sema, i + 2)
          wait_gather(eidxb, rowsb, semb)
          scatter(eidxb, rowsb)

    plsc.subcore_barrier()

    @pl.when(c == 0)
    def _():
      pltpu.sync_copy(acc.at[pl.ds(rbase, rps)], pa_hbm.at[pl.ds(rbase, rps)])

    @pl.when(c == 1)
    def _():
      pltpu.sync_copy(acc.at[pl.ds(rbase, rps)], pb_hbm.at[pl.ds(rbase, rps)])

    if with_cnt:
      pltpu.sync_copy(cacc.at[pl.ds(rbase, rps)],
                      cnt_hbm.at[c].at[pl.ds(rbase, rps)])

  return pl.kernel(body, out_type=tuple(out_type), mesh=mesh,
                   scratch_types=scratch)


def _dense_body(p0, p1, c0, c1, w1, b1r, w2, out):
  s = p0[...] + p1[...]
  cc = c0[...] + c1[...]
  inv = jnp.where(cc > 0.0, 1.0 / cc, 0.0)
  h = lax.dot_general(s * inv, w1[...], (((1,), (1,)), ((), ())),
                      preferred_element_type=jnp.float32)
  h = jnp.maximum(h + b1r[...], 0.0)
  out[...] = lax.dot_general(h, w2[...], (((1,), (1,)), ((), ())),
                             preferred_element_type=jnp.float32)


def _finish_body(p0, p1, c0, c1, b2r, out):
  s = p0[...] + p1[...]
  cc = c0[...] + c1[...]
  inv = jnp.where(cc > 0.0, 1.0 / cc, 0.0)
  r = s * inv + b2r[...]
  nrm = jnp.sqrt(jnp.sum(r * r, axis=1, keepdims=True))
  out[...] = r / jnp.maximum(nrm, 1e-12)


def _row_specs():
  return [
      pl.BlockSpec((BM, D), lambda i: (i, 0)),
      pl.BlockSpec((BM, D), lambda i: (i, 0)),
      pl.BlockSpec((BM, 1), lambda i: (i, 0)),
      pl.BlockSpec((BM, 1), lambda i: (i, 0)),
  ]


@jax.jit
def kernel(x, edge_index, W1, b1, W2, b2):
  e = edge_index.shape[1]
  e_pad = -(-e // CHUNK) * CHUNK
  if e_pad != e:
    pad = jnp.concatenate(
        [jnp.zeros((1, e_pad - e), jnp.int32),
         jnp.full((1, e_pad - e), N, jnp.int32)])
    edge_index = jnp.concatenate([edge_index, pad], axis=1)
  zr = jnp.zeros((NPAD // NS, D), jnp.float32)
  zc = jnp.zeros((NPAD // NS,), jnp.float32)

  p1a, p1b, cnt = _make_spmm(N, e_pad, True)(x, edge_index, zr, zc)
  c0 = cnt[0][:, None]
  c1 = cnt[1][:, None]

  grid = (N // BM,)
  b = pl.pallas_call(
      _dense_body,
      grid=grid,
      in_specs=_row_specs() + [
          pl.BlockSpec((D_HID, D), lambda i: (0, 0)),
          pl.BlockSpec((1, D_HID), lambda i: (0, 0)),
          pl.BlockSpec((D, D_HID), lambda i: (0, 0)),
      ],
      out_specs=pl.BlockSpec((BM, D), lambda i: (i, 0)),
      out_shape=jax.ShapeDtypeStruct((N, D), jnp.float32),
  )(p1a, p1b, c0, c1, W1, b1[None, :], W2)

  p2a, p2b = _make_spmm(N, e_pad, False)(b, edge_index, zr)

  out = pl.pallas_call(
      _finish_body,
      grid=grid,
      in_specs=_row_specs() + [pl.BlockSpec((1, D), lambda i: (0, 0))],
      out_specs=pl.BlockSpec((BM, D), lambda i: (i, 0)),
      out_shape=jax.ShapeDtypeStruct((N, D), jnp.float32),
  )(p2a, p2b, c0, c1, b2[None, :])
  return out
